# Initial kernel scaffold; baseline (speedup 1.0000x reference)
import jax
import jax.numpy as jnp
from jax.experimental import pallas as pl


def kernel(y, E):
    raise NotImplementedError("write your pallas kernel here")


# speedup vs baseline: 4.8065x; 4.8065x over previous
import functools

import jax
import jax.numpy as jnp
from jax import lax
from jax.experimental import pallas as pl
from jax.experimental.pallas import tpu as pltpu
from jax.experimental.pallas import tpu_sc as plsc

_IDX_W = 128
_K = 8
_CHUNK = _K * _IDX_W


@functools.cache
def _build(V, D, B):
    try:
        info = plsc.get_sparse_core_info()
        NC, NS = info.num_cores, info.num_subcores
    except Exception:
        NC, NS = 2, 16
    NW = NC * NS
    assert B % (NW * _CHUNK) == 0
    b_per_w = B // NW
    n_chunks = b_per_w // _CHUNK

    mesh = plsc.VectorSubcoreMesh(core_axis_name="c", subcore_axis_name="s")

    @functools.partial(
        pl.kernel,
        mesh=mesh,
        compiler_params=pltpu.CompilerParams(use_tc_tiling_on_sc=False),
        out_type=jax.ShapeDtypeStruct((B, D), jnp.float32),
        scratch_types=[
            pltpu.VMEM((_K, _IDX_W), jnp.int32),
            pltpu.VMEM((_CHUNK, D), jnp.float32),
            pltpu.SemaphoreType.DMA,
        ],
    )
    def emb(table_hbm, idx_hbm, out_hbm, idx_v, rows_v, sem):
        wid = lax.axis_index("s") * NC + lax.axis_index("c")
        w_row0 = wid * (b_per_w // _IDX_W)

        def chunk_body(g, _):
            row0 = w_row0 + g * _K
            pltpu.sync_copy(idx_hbm.at[pl.ds(row0, _K)], idx_v)
            copies = [
                pltpu.async_copy(
                    table_hbm.at[idx_v.at[j]],
                    rows_v.at[pl.ds(j * _IDX_W, _IDX_W)],
                    sem,
                )
                for j in range(_K)
            ]
            for c in copies:
                c.wait()
            pltpu.sync_copy(rows_v, out_hbm.at[pl.ds(row0 * _IDX_W, _CHUNK)])
            return 0

        lax.fori_loop(0, n_chunks, chunk_body, 0)

    return emb


def kernel(y, E):
    Bt, H = y.shape
    V, D = E.shape
    B = Bt * H
    idx2d = y.reshape(B // _IDX_W, _IDX_W)
    out = _build(V, D, B)(E, idx2d)
    return out.reshape(Bt, H, D)


# speedup vs baseline: 5.0337x; 1.0473x over previous
import functools

import jax
import jax.numpy as jnp
from jax import lax
from jax.experimental import pallas as pl
from jax.experimental.pallas import tpu as pltpu
from jax.experimental.pallas import tpu_sc as plsc

_IDX_W = 128
_K = 8
_CHUNK = _K * _IDX_W
_NBUF = 2


@functools.cache
def _build(V, D, B):
    try:
        info = plsc.get_sparse_core_info()
        NC, NS = info.num_cores, info.num_subcores
    except Exception:
        NC, NS = 2, 16
    NW = NC * NS
    assert B % (NW * _CHUNK * _NBUF) == 0
    b_per_w = B // NW
    n_chunks = b_per_w // _CHUNK

    mesh = plsc.VectorSubcoreMesh(core_axis_name="c", subcore_axis_name="s")

    @functools.partial(
        pl.kernel,
        mesh=mesh,
        compiler_params=pltpu.CompilerParams(use_tc_tiling_on_sc=False),
        out_type=jax.ShapeDtypeStruct((B, D), jnp.float32),
        scratch_types=[
            pltpu.VMEM((_NBUF, _K, _IDX_W), jnp.int32),
            pltpu.VMEM((_NBUF, _CHUNK, D), jnp.float32),
            pltpu.SemaphoreType.DMA,
            pltpu.SemaphoreType.DMA,
            pltpu.SemaphoreType.DMA,
        ],
    )
    def emb(table_hbm, idx_hbm, out_hbm, idx_v, rows_v, sem_idx, sem_g, sem_o):
        wid = lax.axis_index("s") * NC + lax.axis_index("c")
        w_row0 = wid * (b_per_w // _IDX_W)

        def idx_start(g, slot):
            pltpu.async_copy(
                idx_hbm.at[pl.ds(w_row0 + g * _K, _K)], idx_v.at[slot], sem_idx
            )

        def idx_wait(slot):
            pltpu.make_async_copy(
                idx_hbm.at[pl.ds(w_row0, _K)], idx_v.at[slot], sem_idx
            ).wait()

        def store_start(g, slot):
            pltpu.async_copy(
                rows_v.at[slot],
                out_hbm.at[pl.ds((w_row0 + g * _K) * _IDX_W, _CHUNK)],
                sem_o,
            )

        def store_wait(slot):
            pltpu.make_async_copy(
                rows_v.at[slot],
                out_hbm.at[pl.ds(w_row0 * _IDX_W, _CHUNK)],
                sem_o,
            ).wait()

        idx_start(0, 0)

        def outer(o, _):
            for b in range(_NBUF):
                g = o * _NBUF + b
                idx_wait(b)

                @pl.when(g + 1 < n_chunks)
                def _():
                    idx_start(g + 1, 1 - b)

                @pl.when(g >= _NBUF)
                def _():
                    store_wait(b)

                copies = [
                    pltpu.async_copy(
                        table_hbm.at[idx_v.at[b].at[j]],
                        rows_v.at[b].at[pl.ds(j * _IDX_W, _IDX_W)],
                        sem_g,
                    )
                    for j in range(_K)
                ]
                for c in copies:
                    c.wait()
                store_start(g, b)
            return 0

        lax.fori_loop(0, n_chunks // _NBUF, outer, 0)
        for b in range(_NBUF):
            store_wait(b)

    return emb


def kernel(y, E):
    Bt, H = y.shape
    V, D = E.shape
    B = Bt * H
    idx2d = y.reshape(B // _IDX_W, _IDX_W)
    out = _build(V, D, B)(E, idx2d)
    return out.reshape(Bt, H, D)
